# Initial kernel scaffold; baseline (speedup 1.0000x reference)
import jax
import jax.numpy as jnp
from jax.experimental import pallas as pl


def kernel(x, router_weight, e_score_correction_bias, w_gate, w_up, w_down):
    raise NotImplementedError("write your pallas kernel here")


# speedup vs baseline: 2.1010x; 2.1010x over previous
import functools

import jax
import jax.numpy as jnp
from jax import lax
from jax.experimental import pallas as pl
from jax.experimental.pallas import tpu as pltpu
from jax.experimental.pallas import tpu_sc as plsc

D = 1024
FF = 512
E = 16
T = 2048
TILE = 128
NT = (T * 2) // TILE + E
P = NT * TILE


def _route_body(x_ref, rw_ref, bias_ref, d0_ref, d1_ref, w0_ref, w1_ref, te_ref):
    x = x_ref[...]
    rw = rw_ref[...]
    logits = lax.dot_general(x, rw, (((1,), (1,)), ((), ())),
                             preferred_element_type=jnp.float32)
    scores = jax.nn.sigmoid(logits)
    biased = scores + bias_ref[...]
    lane = lax.broadcasted_iota(jnp.int32, (T, E), 1)

    m1 = jnp.max(biased, axis=1, keepdims=True)
    i1 = jnp.min(jnp.where(biased == m1, lane, E), axis=1, keepdims=True)
    sel1 = lane == i1
    w1r = jnp.sum(jnp.where(sel1, scores, 0.0), axis=1, keepdims=True)

    biased2 = jnp.where(sel1, -1e30, biased)
    m2 = jnp.max(biased2, axis=1, keepdims=True)
    i2 = jnp.min(jnp.where(biased2 == m2, lane, E), axis=1, keepdims=True)
    sel2 = lane == i2
    w2r = jnp.sum(jnp.where(sel2, scores, 0.0), axis=1, keepdims=True)

    denom = jnp.clip(w1r + w2r, 1e-20, None)
    w0_ref[...] = jnp.broadcast_to(w1r / denom, (T, 16))
    w1_ref[...] = jnp.broadcast_to(w2r / denom, (T, 16))

    onehot = sel1.astype(jnp.float32) + sel2.astype(jnp.float32)
    c = onehot
    sh = 1
    while sh < T:
        c = c + jnp.concatenate(
            [jnp.zeros((sh, E), jnp.float32), c[:-sh, :]], axis=0)
        sh *= 2
    rank = c - onehot
    total = c[T - 1:T, :]
    padded = (((total.astype(jnp.int32) + (TILE - 1)) // TILE) * TILE)

    er = lax.broadcasted_iota(jnp.int32, (E, E), 0)
    ec = lax.broadcasted_iota(jnp.int32, (E, E), 1)
    maskf = (er < ec).astype(jnp.float32)
    offs = lax.dot_general(padded.astype(jnp.float32), maskf,
                           (((1,), (0,)), ((), ())),
                           preferred_element_type=jnp.float32)

    pos = offs + rank
    d0_ref[...] = jnp.sum(jnp.where(sel1, pos, 0.0), axis=1,
                          keepdims=True).astype(jnp.int32)
    d1_ref[...] = jnp.sum(jnp.where(sel2, pos, 0.0), axis=1,
                          keepdims=True).astype(jnp.int32)

    jt = lax.broadcasted_iota(jnp.int32, (NT, E), 0) * TILE
    offs_i = offs.astype(jnp.int32)
    cnt = jnp.sum((offs_i <= jt).astype(jnp.int32), axis=1, keepdims=True) - 1
    te_ref[...] = jnp.maximum(cnt, 0)


def _route(x2, rw, bias2):
    return pl.pallas_call(
        _route_body,
        out_shape=(
            jax.ShapeDtypeStruct((T, 1), jnp.int32),
            jax.ShapeDtypeStruct((T, 1), jnp.int32),
            jax.ShapeDtypeStruct((T, 16), jnp.float32),
            jax.ShapeDtypeStruct((T, 16), jnp.float32),
            jax.ShapeDtypeStruct((NT, 1), jnp.int32),
        ),
    )(x2, rw, bias2)


def _gemm_body(te_ref, x_ref, wg_ref, wu_ref, wd_ref, y_ref):
    del te_ref
    x = x_ref[...]
    gate = lax.dot_general(x, wg_ref[0], (((1,), (1,)), ((), ())),
                           preferred_element_type=jnp.float32)
    up = lax.dot_general(x, wu_ref[0], (((1,), (1,)), ((), ())),
                         preferred_element_type=jnp.float32)
    h = gate * jax.nn.sigmoid(gate) * up
    y_ref[...] = lax.dot_general(h, wd_ref[0], (((1,), (1,)), ((), ())),
                                 preferred_element_type=jnp.float32)


def _gemm(te, x_sorted, w_gate, w_up, w_down):
    grid_spec = pltpu.PrefetchScalarGridSpec(
        num_scalar_prefetch=1,
        grid=(NT,),
        in_specs=[
            pl.BlockSpec((TILE, D), lambda j, te: (j, 0)),
            pl.BlockSpec((1, FF, D), lambda j, te: (te[j], 0, 0)),
            pl.BlockSpec((1, FF, D), lambda j, te: (te[j], 0, 0)),
            pl.BlockSpec((1, D, FF), lambda j, te: (te[j], 0, 0)),
        ],
        out_specs=pl.BlockSpec((TILE, D), lambda j, te: (j, 0)),
    )
    return pl.pallas_call(
        _gemm_body,
        grid_spec=grid_spec,
        out_shape=jax.ShapeDtypeStruct((P, D), jnp.float32),
    )(te, x_sorted, w_gate, w_up, w_down)


def _sc_scatter(x2, d0, d1):
    info = plsc.get_sparse_core_info()
    nw = info.num_cores * info.num_subcores
    ch = T // nw
    mesh = plsc.VectorSubcoreMesh(core_axis_name="c", subcore_axis_name="s")

    @functools.partial(
        pl.kernel,
        out_type=jax.ShapeDtypeStruct((P, D), jnp.float32),
        mesh=mesh,
        scratch_types=[
            pltpu.VMEM((ch, D), jnp.float32),
            pltpu.VMEM((ch,), jnp.int32),
            pltpu.VMEM((ch,), jnp.int32),
            pltpu.SemaphoreType.DMA,
            pltpu.SemaphoreType.DMA,
        ],
    )
    def k(x_hbm, d0_hbm, d1_hbm, out_hbm, rows_v, i0_v, i1_v, s0, s1):
        wid = lax.axis_index("s") * info.num_cores + lax.axis_index("c")
        base = wid * ch
        pltpu.sync_copy(x_hbm.at[pl.ds(base, ch)], rows_v)
        pltpu.sync_copy(d0_hbm.at[pl.ds(base, ch)], i0_v)
        pltpu.sync_copy(d1_hbm.at[pl.ds(base, ch)], i1_v)
        c0 = pltpu.async_copy(rows_v, out_hbm.at[i0_v], s0)
        c1 = pltpu.async_copy(rows_v, out_hbm.at[i1_v], s1)
        c0.wait()
        c1.wait()

    return k(x2, d0, d1)


def _sc_combine(y, d0, d1, w0, w1):
    info = plsc.get_sparse_core_info()
    nw = info.num_cores * info.num_subcores
    per_w = T // nw
    ch = 32
    nch = per_w // ch
    mesh = plsc.VectorSubcoreMesh(core_axis_name="c", subcore_axis_name="s")

    @functools.partial(
        pl.kernel,
        out_type=jax.ShapeDtypeStruct((T, D), jnp.float32),
        mesh=mesh,
        scratch_types=[
            pltpu.VMEM((ch, D), jnp.float32),
            pltpu.VMEM((ch, D), jnp.float32),
            pltpu.VMEM((ch,), jnp.int32),
            pltpu.VMEM((ch,), jnp.int32),
            pltpu.VMEM((ch, 16), jnp.float32),
            pltpu.VMEM((ch, 16), jnp.float32),
            pltpu.SemaphoreType.DMA,
            pltpu.SemaphoreType.DMA,
        ],
    )
    def k(y_hbm, d0_hbm, d1_hbm, w0_hbm, w1_hbm, out_hbm,
          r0_v, r1_v, i0_v, i1_v, wa_v, wb_v, s0, s1):
        wid = lax.axis_index("s") * info.num_cores + lax.axis_index("c")
        for c in range(nch):
            base = wid * per_w + c * ch
            pltpu.sync_copy(d0_hbm.at[pl.ds(base, ch)], i0_v)
            pltpu.sync_copy(d1_hbm.at[pl.ds(base, ch)], i1_v)
            pltpu.sync_copy(w0_hbm.at[pl.ds(base, ch)], wa_v)
            pltpu.sync_copy(w1_hbm.at[pl.ds(base, ch)], wb_v)
            g0 = pltpu.async_copy(y_hbm.at[i0_v], r0_v, s0)
            g1 = pltpu.async_copy(y_hbm.at[i1_v], r1_v, s1)
            g0.wait()
            g1.wait()

            def tok_body(i, _):
                sa = wa_v[i, :]
                sb = wb_v[i, :]

                def vec_body(j, _):
                    sl = pl.ds(j * 16, 16)
                    r0_v[i, sl] = r0_v[i, sl] * sa + r1_v[i, sl] * sb
                    return 0

                return lax.fori_loop(0, D // 16, vec_body, 0)

            lax.fori_loop(0, ch, tok_body, 0)
            pltpu.sync_copy(r0_v, out_hbm.at[pl.ds(base, ch)])

    return k(y, d0, d1, w0, w1)


def kernel(x, router_weight, e_score_correction_bias, w_gate, w_up, w_down):
    x2 = x.reshape(T, D)
    bias2 = e_score_correction_bias.reshape(1, E)
    d0, d1, w0, w1, te = _route(x2, router_weight, bias2)
    d0 = d0.reshape(T)
    d1 = d1.reshape(T)
    te = te.reshape(NT)
    x_sorted = _sc_scatter(x2, d0, d1)
    y = _gemm(te, x_sorted, w_gate, w_up, w_down)
    out = _sc_combine(y, d0, d1, w0, w1)
    return out.reshape(1, T, D)
